# Initial kernel scaffold; baseline (speedup 1.0000x reference)
import jax
import jax.numpy as jnp
from jax.experimental import pallas as pl


def kernel(strength, length, phrase, Ws, Wl, Wp, W_proj, b_proj):
    raise NotImplementedError("write your pallas kernel here")


# speedup vs baseline: 1.3948x; 1.3948x over previous
import functools
import math

import numpy as np
import jax
import jax.numpy as jnp
from jax import lax
from jax.experimental import pallas as pl
from jax.experimental.pallas import tpu as pltpu
from jax.experimental.pallas import tpu_sc as plsc

_B, _L, _D = 16, 512, 512
_T = _B * _L
_NW = 32
_TPW = _T // _NW
_CH = 32
_NCH = _TPW // _CH
_VPC = _CH * (_D // 16)


def _pos_enc(seq_len: int, d: int) -> np.ndarray:
    channels = int(math.ceil(d / 2) * 2)
    inv_freq = 1.0 / (10000 ** (np.arange(0, channels, 2, dtype=np.float32) / channels))
    pos = np.arange(seq_len, dtype=np.float32)
    sin_inp = np.einsum("i,j->ij", pos, inv_freq.astype(np.float32))
    emb = np.stack((np.sin(sin_inp), np.cos(sin_inp)), axis=-1).reshape(seq_len, channels)
    return emb[:, :d].astype(np.float32)


def _fold_body(ws_ref, wl_ref, wp_ref, wproj_ref, b_ref, ps_ref, pl_ref, pp_ref):
    b = b_ref[...]
    ps_ref[...] = jnp.dot(ws_ref[...], wproj_ref[0:_D, :],
                          preferred_element_type=jnp.float32) + b
    pl_ref[...] = jnp.dot(wl_ref[...], wproj_ref[_D:2 * _D, :],
                          preferred_element_type=jnp.float32)
    pp_ref[...] = jnp.dot(wp_ref[...], wproj_ref[2 * _D:3 * _D, :],
                          preferred_element_type=jnp.float32)


_fold_tables = pl.pallas_call(
    _fold_body,
    out_shape=(
        jax.ShapeDtypeStruct((16, _D), jnp.float32),
        jax.ShapeDtypeStruct((32, _D), jnp.float32),
        jax.ShapeDtypeStruct((64, _D), jnp.float32),
    ),
)


def _sc_body(s_hbm, l_hbm, p_hbm, ps_hbm, pl_hbm, pp_hbm, pe_hbm, out_hbm,
             s_idx, l_idx, p_idx, buf_s, buf_l, buf_p, obuf,
             sem_s, sem_l, sem_p):
    wid = lax.axis_index("s") * 2 + lax.axis_index("c")
    base = wid * _TPW

    def chunk_body(ci, carry):
        tb = base + ci * _CH
        pltpu.sync_copy(s_hbm.at[pl.ds(tb, _CH)], s_idx)
        pltpu.sync_copy(l_hbm.at[pl.ds(tb, _CH)], l_idx)
        pltpu.sync_copy(p_hbm.at[pl.ds(tb, _CH)], p_idx)
        cs = pltpu.async_copy(ps_hbm.at[s_idx], buf_s, sem_s)
        cl = pltpu.async_copy(pl_hbm.at[l_idx], buf_l, sem_l)
        cp = pltpu.async_copy(pp_hbm.at[p_idx], buf_p, sem_p)
        pb = lax.rem(tb, _L)
        pltpu.sync_copy(pe_hbm.at[pl.ds(pb, _CH)], obuf)
        cs.wait()
        cl.wait()
        cp.wait()

        def vbody(i, c):
            r = i // (_D // 16)
            col = (i % (_D // 16)) * 16
            sl = pl.ds(col, 16)
            obuf[r, sl] = obuf[r, sl] + (buf_s[r, sl] + buf_l[r, sl]
                                         + buf_p[r, sl])
            return c

        lax.fori_loop(0, _VPC, vbody, 0, unroll=8)
        pltpu.sync_copy(obuf, out_hbm.at[pl.ds(tb, _CH)])
        return carry

    lax.fori_loop(0, _NCH, chunk_body, 0)


_sc_gather = functools.partial(
    pl.kernel,
    out_type=jax.ShapeDtypeStruct((_T, _D), jnp.float32),
    mesh=plsc.VectorSubcoreMesh(core_axis_name="c", subcore_axis_name="s"),
    scratch_types=[
        pltpu.VMEM((_CH,), jnp.int32),
        pltpu.VMEM((_CH,), jnp.int32),
        pltpu.VMEM((_CH,), jnp.int32),
        pltpu.VMEM((_CH, _D), jnp.float32),
        pltpu.VMEM((_CH, _D), jnp.float32),
        pltpu.VMEM((_CH, _D), jnp.float32),
        pltpu.VMEM((_CH, _D), jnp.float32),
        pltpu.SemaphoreType.DMA,
        pltpu.SemaphoreType.DMA,
        pltpu.SemaphoreType.DMA,
    ],
)(_sc_body)

_PE = _pos_enc(_L, _D)


@jax.jit
def _run(strength, length, phrase, Ws, Wl, Wp, W_proj, b_proj):
    s = strength.reshape(_T).astype(jnp.int32)
    l = length.reshape(_T).astype(jnp.int32)
    p = phrase.reshape(_T).astype(jnp.int32)
    ps, pl_t, pp = _fold_tables(Ws, Wl, Wp, W_proj, b_proj.reshape(1, _D))
    pe = jnp.asarray(_PE)
    out = _sc_gather(s, l, p, ps, pl_t, pp, pe)
    return out.reshape(_B, _L, _D)


def kernel(strength, length, phrase, Ws, Wl, Wp, W_proj, b_proj):
    return _run(strength, length, phrase, Ws, Wl, Wp, W_proj, b_proj)


# speedup vs baseline: 1.4533x; 1.0419x over previous
import functools
import math

import numpy as np
import jax
import jax.numpy as jnp
from jax import lax
from jax.experimental import pallas as pl
from jax.experimental.pallas import tpu as pltpu
from jax.experimental.pallas import tpu_sc as plsc

_B, _L, _D = 16, 512, 512
_NW = 32
_PB = _L // _NW
_NV = _D // 16


def _pos_enc(seq_len: int, d: int) -> np.ndarray:
    channels = int(math.ceil(d / 2) * 2)
    inv_freq = 1.0 / (10000 ** (np.arange(0, channels, 2, dtype=np.float32) / channels))
    pos = np.arange(seq_len, dtype=np.float32)
    sin_inp = np.einsum("i,j->ij", pos, inv_freq.astype(np.float32))
    emb = np.stack((np.sin(sin_inp), np.cos(sin_inp)), axis=-1).reshape(seq_len, channels)
    return emb[:, :d].astype(np.float32)


def _fold_body(ws_ref, wl_ref, wp_ref, wproj_ref, b_ref, ps_ref, pl_ref, pp_ref):
    b = b_ref[...]
    ps_ref[...] = jnp.dot(ws_ref[...], wproj_ref[0:_D, :],
                          preferred_element_type=jnp.float32) + b
    pl_ref[...] = jnp.dot(wl_ref[...], wproj_ref[_D:2 * _D, :],
                          preferred_element_type=jnp.float32)
    pp_ref[...] = jnp.dot(wp_ref[...], wproj_ref[2 * _D:3 * _D, :],
                          preferred_element_type=jnp.float32)


_fold_tables = pl.pallas_call(
    _fold_body,
    out_shape=(
        jax.ShapeDtypeStruct((16, _D), jnp.float32),
        jax.ShapeDtypeStruct((32, _D), jnp.float32),
        jax.ShapeDtypeStruct((64, _D), jnp.float32),
    ),
)


def _sc_body(s_hbm, l_hbm, p_hbm, ps_hbm, pl_hbm, pp_hbm, pe_hbm, out_hbm,
             s_idx, l_idx, p_idx, pe_b,
             bs0, bl0, bp0, bs1, bl1, bp1, ob0, ob1,
             sem_g0, sem_g1, sem_o0, sem_o1):
    wid = lax.axis_index("s") * 2 + lax.axis_index("c")
    colbase = wid * _PB
    ibase = wid * (_B * _PB)

    pltpu.sync_copy(s_hbm.at[pl.ds(ibase, _B * _PB)], s_idx)
    pltpu.sync_copy(l_hbm.at[pl.ds(ibase, _B * _PB)], l_idx)
    pltpu.sync_copy(p_hbm.at[pl.ds(ibase, _B * _PB)], p_idx)
    pltpu.sync_copy(pe_hbm.at[pl.ds(colbase, _PB)], pe_b)

    gsets = ((bs0, bl0, bp0, sem_g0), (bs1, bl1, bp1, sem_g1))
    osets = ((ob0, sem_o0), (ob1, sem_o1))

    def g_copies(b, which):
        bs, bl, bp, sg = gsets[which]
        sl = pl.ds(b * _PB, _PB)
        return (pltpu.make_async_copy(ps_hbm.at[s_idx.at[sl]], bs, sg),
                pltpu.make_async_copy(pl_hbm.at[l_idx.at[sl]], bl, sg),
                pltpu.make_async_copy(pp_hbm.at[p_idx.at[sl]], bp, sg))

    def g_start(b, which):
        for c in g_copies(b, which):
            c.start()

    def g_wait(b, which):
        for c in g_copies(b, which):
            c.wait()

    def o_copy(b, which):
        ob, so = osets[which]
        return pltpu.make_async_copy(ob, out_hbm.at[b, pl.ds(colbase, _PB)],
                                     so)

    def compute(which):
        bs, bl, bp, _ = gsets[which]
        ob, _ = osets[which]

        def tok(j, c):
            for c32 in range(_NV):
                sl = pl.ds(c32 * 16, 16)
                ob[j, sl] = (bs[j, sl] + bl[j, sl] + bp[j, sl]
                             + pe_b[j, sl])
            return c

        lax.fori_loop(0, _PB, tok, 0)

    def chunk(b, which, drain, prefetch):
        g_wait(b, which)
        if drain:
            o_copy(b - 2, which).wait()
        compute(which)
        if prefetch:
            g_start(b + 2, which)
        o_copy(b, which).start()

    g_start(0, 0)
    g_start(1, 1)
    chunk(0, 0, drain=False, prefetch=True)
    chunk(1, 1, drain=False, prefetch=True)

    def pair(i, c):
        b = 2 * i
        chunk(b, 0, drain=True, prefetch=True)
        chunk(b + 1, 1, drain=True, prefetch=True)
        return c

    lax.fori_loop(1, _B // 2 - 1, pair, 0)
    chunk(_B - 2, 0, drain=True, prefetch=False)
    chunk(_B - 1, 1, drain=True, prefetch=False)
    o_copy(_B - 2, 0).wait()
    o_copy(_B - 1, 1).wait()


_sc_gather = functools.partial(
    pl.kernel,
    out_type=jax.ShapeDtypeStruct((_B, _L, _D), jnp.float32),
    mesh=plsc.VectorSubcoreMesh(core_axis_name="c", subcore_axis_name="s"),
    scratch_types=[
        pltpu.VMEM((_B * _PB,), jnp.int32),
        pltpu.VMEM((_B * _PB,), jnp.int32),
        pltpu.VMEM((_B * _PB,), jnp.int32),
        pltpu.VMEM((_PB, _D), jnp.float32),
        pltpu.VMEM((_PB, _D), jnp.float32),
        pltpu.VMEM((_PB, _D), jnp.float32),
        pltpu.VMEM((_PB, _D), jnp.float32),
        pltpu.VMEM((_PB, _D), jnp.float32),
        pltpu.VMEM((_PB, _D), jnp.float32),
        pltpu.VMEM((_PB, _D), jnp.float32),
        pltpu.VMEM((_PB, _D), jnp.float32),
        pltpu.VMEM((_PB, _D), jnp.float32),
        pltpu.SemaphoreType.DMA,
        pltpu.SemaphoreType.DMA,
        pltpu.SemaphoreType.DMA,
        pltpu.SemaphoreType.DMA,
    ],
)(_sc_body)

_PE = _pos_enc(_L, _D)


@jax.jit
def _run(strength, length, phrase, Ws, Wl, Wp, W_proj, b_proj):
    def _band(x):
        return (x.astype(jnp.int32).reshape(_B, _NW, _PB)
                .transpose(1, 0, 2).reshape(_NW * _B * _PB))

    s = _band(strength)
    l = _band(length)
    p = _band(phrase)
    ps, pl_t, pp = _fold_tables(Ws, Wl, Wp, W_proj, b_proj.reshape(1, _D))
    pe = jnp.asarray(_PE)
    return _sc_gather(s, l, p, ps, pl_t, pp, pe)


def kernel(strength, length, phrase, Ws, Wl, Wp, W_proj, b_proj):
    return _run(strength, length, phrase, Ws, Wl, Wp, W_proj, b_proj)
